# Initial kernel scaffold; baseline (speedup 1.0000x reference)
import jax
import jax.numpy as jnp
from jax.experimental import pallas as pl


def kernel(atomic_numbers, pos, edge_index, batch, W0, b0, W1, b1, W2, b2, Wl, bl):
    raise NotImplementedError("write your pallas kernel here")


# speedup vs baseline: 4.3461x; 4.3461x over previous
import functools

import jax
import jax.numpy as jnp
from jax import lax
from jax.experimental import pallas as pl
from jax.experimental.pallas import tpu as pltpu
from jax.experimental.pallas import tpu_sc as plsc

N = 50000
E = 800000
H = 64
HS = 8
NS = H // HS
NX = 8
G = 512

NC = 2
TILES = 16
NW = NC * TILES

NPAD = 50176
EPAD = 802816
CH = 128
NCH_L = EPAD // TILES // CH
NCH_D = EPAD // NW // CH
ROWS_PT = NPAD // TILES

NODES_PT = NPAD // NW
CHP = 112
NCHP = NODES_PT // CHP
GPAD = G + 8

BN = 512
GRID = NPAD // BN

_F32S = jax.ShapeDtypeStruct((NPAD, HS), jnp.float32)


@functools.cache
def _sc_mesh():
    return plsc.VectorSubcoreMesh(
        core_axis_name="c", subcore_axis_name="s",
        num_cores=NC, num_subcores=TILES,
    )


@functools.cache
def _deg_kernel():
    return functools.partial(
        pl.kernel,
        out_type=[
            jax.ShapeDtypeStruct((NPAD, 1), jnp.float32),
            jax.ShapeDtypeStruct((NPAD, 1), jnp.float32),
        ],
        mesh=_sc_mesh(),
        compiler_params=pltpu.CompilerParams(use_tc_tiling_on_sc=False),
        scratch_types=[
            pltpu.VMEM((NCH_D, CH), jnp.int32),
            pltpu.VMEM((CH, 1), jnp.float32),
            pltpu.VMEM_SHARED((NPAD, 1), jnp.float32),
        ],
    )(_deg_body)


def _deg_call(dst_d, ones_ch, zcol):
    return _deg_kernel()(dst_d, ones_ch, zcol)


def _deg_body(dst_d, ones_ch, zcol, out0, out1, dstb, onesb, acc):
    c = lax.axis_index("c")
    s = lax.axis_index("s")
    w = c * TILES + s
    r0 = s * ROWS_PT
    pltpu.sync_copy(zcol, acc.at[pl.ds(r0, ROWS_PT)])
    pltpu.sync_copy(dst_d.at[w], dstb)
    pltpu.sync_copy(ones_ch, onesb)
    plsc.subcore_barrier()

    def chunk(j, carry):
        pltpu.sync_copy(onesb, acc.at[dstb.at[j]], add=True)
        return carry

    lax.fori_loop(0, NCH_D, chunk, 0)
    plsc.subcore_barrier()

    @pl.when(c == 0)
    def _():
        pltpu.sync_copy(acc.at[pl.ds(r0, ROWS_PT)], out0.at[pl.ds(r0, ROWS_PT)])

    @pl.when(c == 1)
    def _():
        pltpu.sync_copy(acc.at[pl.ds(r0, ROWS_PT)], out1.at[pl.ds(r0, ROWS_PT)])


@functools.cache
def _agg_kernel():
    return functools.partial(
        pl.kernel,
        out_type=[_F32S] * NS,
        mesh=_sc_mesh(),
        compiler_params=pltpu.CompilerParams(use_tc_tiling_on_sc=False),
        scratch_types=[
            pltpu.VMEM((NCH_L, CH), jnp.int32),
            pltpu.VMEM((NCH_L, CH), jnp.int32),
            pltpu.VMEM((CH, HS), jnp.float32),
            pltpu.SemaphoreType.DMA,
            pltpu.VMEM_SHARED((NPAD, HS), jnp.float32),
        ],
    )(_agg_body)


def _agg_call(yq, src0_l, src1_l, dst_l, zrows):
    yp = [jnp.concatenate([yq[2 * p], yq[2 * p + 1]], axis=0)
          for p in range(NS // NC)]
    return _agg_kernel()(*yp, src0_l, src1_l, dst_l, zrows)


def _agg_body(*refs):
    np_ = NS // NC
    yp = refs[:np_]
    src0_l, src1_l, dst_l, zrows = refs[np_:np_ + 4]
    outs = refs[np_ + 4:np_ + 4 + NS]
    srcb, dstb, rows, sem, acc = refs[np_ + 4 + NS:]
    c = lax.axis_index("c")
    s = lax.axis_index("s")
    r0 = s * ROWS_PT

    @pl.when(c == 0)
    def _():
        pltpu.sync_copy(src0_l.at[s], srcb)

    @pl.when(c == 1)
    def _():
        pltpu.sync_copy(src1_l.at[s], srcb)

    pltpu.sync_copy(dst_l.at[s], dstb)

    for p in range(np_):
        pltpu.sync_copy(zrows, acc.at[pl.ds(r0, ROWS_PT)])
        plsc.subcore_barrier()

        def chunk(j, carry, p=p):
            pltpu.async_copy(yp[p].at[srcb.at[j]], rows, sem).wait()
            pltpu.sync_copy(rows, acc.at[dstb.at[j]], add=True)
            return carry

        lax.fori_loop(0, NCH_L, chunk, 0)
        plsc.subcore_barrier()

        @pl.when(c == 0)
        def _():
            pltpu.sync_copy(acc.at[pl.ds(r0, ROWS_PT)],
                            outs[2 * p].at[pl.ds(r0, ROWS_PT)])

        @pl.when(c == 1)
        def _():
            pltpu.sync_copy(acc.at[pl.ds(r0, ROWS_PT)],
                            outs[2 * p + 1].at[pl.ds(r0, ROWS_PT)])


@functools.cache
def _pool_kernel():
    return functools.partial(
        pl.kernel,
        out_type=[
            jax.ShapeDtypeStruct((GPAD, H), jnp.float32),
            jax.ShapeDtypeStruct((GPAD, H), jnp.float32),
            jax.ShapeDtypeStruct((GPAD, 1), jnp.float32),
            jax.ShapeDtypeStruct((GPAD, 1), jnp.float32),
        ],
        mesh=_sc_mesh(),
        compiler_params=pltpu.CompilerParams(use_tc_tiling_on_sc=False),
        scratch_types=[
            pltpu.VMEM((NCHP, CHP), jnp.int32),
            pltpu.VMEM((NCHP, CHP, 1), jnp.float32),
            pltpu.VMEM((CHP, H), jnp.float32),
            pltpu.VMEM_SHARED((GPAD, H), jnp.float32),
            pltpu.VMEM_SHARED((GPAD, 1), jnp.float32),
        ],
    )(_pool_body)


def _pool_call(x3, bidx, ones_n, zg, zg1):
    return _pool_kernel()(x3, bidx, ones_n, zg, zg1)


def _pool_body(x3, bidx, ones_n, zg, zg1, p0, p1, c0, c1,
               idxb, oneb, valb, acc, cacc):
    c = lax.axis_index("c")
    s = lax.axis_index("s")
    w = c * TILES + s
    base = w * NODES_PT

    @pl.when(s == 0)
    def _():
        pltpu.sync_copy(zg, acc)
        pltpu.sync_copy(zg1, cacc)

    pltpu.sync_copy(bidx.at[w], idxb)
    pltpu.sync_copy(ones_n.at[w], oneb)
    plsc.subcore_barrier()

    def chunk(j, carry):
        pltpu.sync_copy(x3.at[pl.ds(base + j * CHP, CHP)], valb)
        pltpu.sync_copy(valb, acc.at[idxb.at[j]], add=True)
        pltpu.sync_copy(oneb.at[j], cacc.at[idxb.at[j]], add=True)
        return carry

    lax.fori_loop(0, NCHP, chunk, 0)
    plsc.subcore_barrier()

    @pl.when(jnp.logical_and(s == 0, c == 0))
    def _():
        pltpu.sync_copy(acc, p0)
        pltpu.sync_copy(cacc, c0)

    @pl.when(jnp.logical_and(s == 0, c == 1))
    def _():
        pltpu.sync_copy(acc, p1)
        pltpu.sync_copy(cacc, c1)


def _tc_pre_body(x4, d0, d1, w0, *outs):
    ys = outs[:NS]
    dis = outs[NS]
    i = pl.program_id(0)
    deg = d0[...] + d1[...] + 1.0
    row = i * BN + lax.broadcasted_iota(jnp.int32, (BN, 1), 0)
    disv = jnp.where(row < N, lax.rsqrt(deg), 0.0)
    h = jnp.dot(x4[...], w0[...], preferred_element_type=jnp.float32)
    y = disv * h
    for k in range(NS):
        ys[k][...] = y[:, k * HS:(k + 1) * HS]
    dis[...] = disv


def _pre_call(x4, d0, d1, W0):
    sspec = pl.BlockSpec((BN, HS), lambda i: (i, 0))
    return pl.pallas_call(
        _tc_pre_body,
        grid=(GRID,),
        in_specs=[
            pl.BlockSpec((BN, 4), lambda i: (i, 0)),
            pl.BlockSpec((BN, 1), lambda i: (i, 0)),
            pl.BlockSpec((BN, 1), lambda i: (i, 0)),
            pl.BlockSpec((4, H), lambda i: (0, 0)),
        ],
        out_specs=[sspec] * NS + [pl.BlockSpec((BN, 1), lambda i: (i, 0))],
        out_shape=[_F32S] * NS + [jax.ShapeDtypeStruct((NPAD, 1), jnp.float32)],
    )(x4, d0, d1, W0)


def _tc_mid_body(*refs):
    aq = refs[:NS]
    yq = refs[NS:2 * NS]
    dis, b, w = refs[2 * NS:2 * NS + 3]
    outs = refs[2 * NS + 3:]
    oq = outs[:NS]
    xout = outs[NS]
    z = jnp.concatenate([aq[k][...] + yq[k][...] for k in range(NS)], axis=1)
    x = jnp.maximum(dis[...] * z + b[...], 0.0)
    h = jnp.dot(x, w[...], preferred_element_type=jnp.float32)
    y = dis[...] * h
    for k in range(NS):
        oq[k][...] = y[:, k * HS:(k + 1) * HS]
    xout[...] = x


def _mid_call(aq, yq, dis, b, W):
    sspec = pl.BlockSpec((BN, HS), lambda i: (i, 0))
    return pl.pallas_call(
        _tc_mid_body,
        grid=(GRID,),
        in_specs=[sspec] * (2 * NS) + [
            pl.BlockSpec((BN, 1), lambda i: (i, 0)),
            pl.BlockSpec((1, H), lambda i: (0, 0)),
            pl.BlockSpec((H, H), lambda i: (0, 0))],
        out_specs=[sspec] * NS + [pl.BlockSpec((BN, H), lambda i: (i, 0))],
        out_shape=[_F32S] * NS + [jax.ShapeDtypeStruct((NPAD, H), jnp.float32)],
    )(*aq, *yq, dis, b, W)


def _tc_gate_body(p0, p1, c0, c1, wl, bl, out):
    pooled = p0[...] + p1[...]
    cnt = c0[...] + c1[...]
    pooled = pooled[:G] / jnp.maximum(cnt[:G], 1.0)
    logits = jnp.dot(pooled, wl[...], preferred_element_type=jnp.float32) + bl[...]
    m = jnp.max(logits, axis=1, keepdims=True)
    e = jnp.exp(logits - m)
    out[...] = e / jnp.sum(e, axis=1, keepdims=True)


def _gate_call(p0, p1, c0, c1, Wl, bl):
    return pl.pallas_call(
        _tc_gate_body,
        out_shape=jax.ShapeDtypeStruct((G, NX), jnp.float32),
    )(p0, p1, c0, c1, Wl, bl)


def kernel(atomic_numbers, pos, edge_index, batch,
           W0, b0, W1, b1, W2, b2, Wl, bl):
    f32 = jnp.float32
    src = edge_index[0]
    dst = edge_index[1]
    pe = EPAD - E
    srcp = jnp.concatenate([src, jnp.full((pe,), N, jnp.int32)])
    dstp = jnp.concatenate([dst, jnp.full((pe,), N, jnp.int32)])
    src0_l = srcp.reshape(TILES, NCH_L, CH)
    src1_l = (srcp + NPAD).reshape(TILES, NCH_L, CH)
    dst_l = dstp.reshape(TILES, NCH_L, CH)
    dst_d = dstp.reshape(NW, NCH_D, CH)

    batchp = jnp.concatenate(
        [batch.astype(jnp.int32), jnp.full((NPAD - N,), G, jnp.int32)]
    ).reshape(NW, NCHP, CHP)
    ones_n = jnp.concatenate(
        [jnp.ones((N,), f32), jnp.zeros((NPAD - N,), f32)]
    ).reshape(NW, NCHP, CHP, 1)

    x4 = jnp.pad(
        jnp.concatenate([atomic_numbers[:, None], pos], axis=1),
        ((0, NPAD - N), (0, 0)),
    )

    zrows = jnp.zeros((ROWS_PT, HS), f32)
    zcol = jnp.zeros((ROWS_PT, 1), f32)
    ones_ch = jnp.ones((CH, 1), f32)
    zg = jnp.zeros((GPAD, H), f32)
    zg1 = jnp.zeros((GPAD, 1), f32)

    d0, d1 = _deg_call(dst_d, ones_ch, zcol)
    *yq, dis = _pre_call(x4, d0, d1, W0)

    Ws = jnp.stack([W1, W2, W1])
    bs = jnp.stack([b0.reshape(1, H), b1.reshape(1, H), b2.reshape(1, H)])
    trip = lax.optimization_barrier(jnp.int32(3))

    def layer_cond(carry):
        return carry[0] < trip

    def layer_step(carry):
        i, yq, _x = carry
        W = lax.dynamic_index_in_dim(Ws, i, keepdims=False)
        b = lax.dynamic_index_in_dim(bs, i, keepdims=False)
        aq = _agg_call(yq, src0_l, src1_l, dst_l, zrows)
        *oq, x = _mid_call(aq, yq, dis, b, W)
        return (i + 1, tuple(oq), x)

    x_init = jnp.zeros((NPAD, H), f32)
    _, yq, x3 = lax.while_loop(
        layer_cond, layer_step, (jnp.int32(0), tuple(yq), x_init))

    p0, p1, c0, c1 = _pool_call(x3, batchp, ones_n, zg, zg1)
    probs = _gate_call(p0, p1, c0, c1, Wl, bl.reshape(1, NX))
    return probs[:, :, None]


# speedup vs baseline: 4.6485x; 1.0696x over previous
import functools

import jax
import jax.numpy as jnp
from jax import lax
from jax.experimental import pallas as pl
from jax.experimental.pallas import tpu as pltpu
from jax.experimental.pallas import tpu_sc as plsc

N = 50000
E = 800000
H = 64
HS = 8
NS = H // HS
NX = 8
G = 512

NC = 2
TILES = 16
NW = NC * TILES

NPAD = 50176
EPAD = 802816
CH = 128
NCH_L = EPAD // TILES // CH
NCH_D = EPAD // NW // CH
ROWS_PT = NPAD // TILES

NODES_PT = NPAD // NW
CHP = 112
NCHP = NODES_PT // CHP
GPAD = G + 8

BN = 512
GRID = NPAD // BN

_F32S = jax.ShapeDtypeStruct((NPAD, HS), jnp.float32)


@functools.cache
def _sc_mesh():
    return plsc.VectorSubcoreMesh(
        core_axis_name="c", subcore_axis_name="s",
        num_cores=NC, num_subcores=TILES,
    )


@functools.cache
def _deg_kernel():
    return functools.partial(
        pl.kernel,
        out_type=[
            jax.ShapeDtypeStruct((NPAD, 1), jnp.float32),
            jax.ShapeDtypeStruct((NPAD, 1), jnp.float32),
        ],
        mesh=_sc_mesh(),
        compiler_params=pltpu.CompilerParams(use_tc_tiling_on_sc=False),
        scratch_types=[
            pltpu.VMEM((NCH_D, CH), jnp.int32),
            pltpu.VMEM((CH, 1), jnp.float32),
            pltpu.VMEM_SHARED((NPAD, 1), jnp.float32),
        ],
    )(_deg_body)


def _deg_call(dst_d, ones_ch, zcol):
    return _deg_kernel()(dst_d, ones_ch, zcol)


def _deg_body(dst_d, ones_ch, zcol, out0, out1, dstb, onesb, acc):
    c = lax.axis_index("c")
    s = lax.axis_index("s")
    w = c * TILES + s
    r0 = s * ROWS_PT
    pltpu.sync_copy(zcol, acc.at[pl.ds(r0, ROWS_PT)])
    pltpu.sync_copy(dst_d.at[w], dstb)
    pltpu.sync_copy(ones_ch, onesb)
    plsc.subcore_barrier()

    def chunk(j, carry):
        pltpu.sync_copy(onesb, acc.at[dstb.at[j]], add=True)
        return carry

    lax.fori_loop(0, NCH_D, chunk, 0)
    plsc.subcore_barrier()

    @pl.when(c == 0)
    def _():
        pltpu.sync_copy(acc.at[pl.ds(r0, ROWS_PT)], out0.at[pl.ds(r0, ROWS_PT)])

    @pl.when(c == 1)
    def _():
        pltpu.sync_copy(acc.at[pl.ds(r0, ROWS_PT)], out1.at[pl.ds(r0, ROWS_PT)])


@functools.cache
def _agg_kernel():
    return functools.partial(
        pl.kernel,
        out_type=[_F32S] * NS,
        mesh=_sc_mesh(),
        compiler_params=pltpu.CompilerParams(use_tc_tiling_on_sc=False),
        scratch_types=[
            pltpu.VMEM((NCH_L, CH), jnp.int32),
            pltpu.VMEM((NCH_L, CH), jnp.int32),
            pltpu.VMEM((CH, HS), jnp.float32),
            pltpu.VMEM((CH, HS), jnp.float32),
            pltpu.SemaphoreType.DMA,
            pltpu.SemaphoreType.DMA,
            pltpu.VMEM_SHARED((NPAD, HS), jnp.float32),
        ],
    )(_agg_body)


def _agg_call(yq, src0_l, src1_l, dst_l, zrows):
    yp = [jnp.concatenate([yq[2 * p], yq[2 * p + 1]], axis=0)
          for p in range(NS // NC)]
    return _agg_kernel()(*yp, src0_l, src1_l, dst_l, zrows)


def _agg_body(*refs):
    np_ = NS // NC
    yp = refs[:np_]
    src0_l, src1_l, dst_l, zrows = refs[np_:np_ + 4]
    outs = refs[np_ + 4:np_ + 4 + NS]
    srcb, dstb, rows0, rows1, sem0, sem1, acc = refs[np_ + 4 + NS:]
    c = lax.axis_index("c")
    s = lax.axis_index("s")
    r0 = s * ROWS_PT

    @pl.when(c == 0)
    def _():
        pltpu.sync_copy(src0_l.at[s], srcb)

    @pl.when(c == 1)
    def _():
        pltpu.sync_copy(src1_l.at[s], srcb)

    pltpu.sync_copy(dst_l.at[s], dstb)

    for p in range(np_):
        pltpu.sync_copy(zrows, acc.at[pl.ds(r0, ROWS_PT)])
        plsc.subcore_barrier()

        pltpu.async_copy(yp[p].at[srcb.at[0]], rows0, sem0)

        def chunk2(t, carry, p=p):
            j0 = 2 * t
            j1 = 2 * t + 1
            j2 = jnp.minimum(2 * t + 2, NCH_L - 1)
            pltpu.make_async_copy(yp[p].at[srcb.at[j0]], rows0, sem0).wait()
            pltpu.async_copy(yp[p].at[srcb.at[j1]], rows1, sem1)
            pltpu.sync_copy(rows0, acc.at[dstb.at[j0]], add=True)
            pltpu.make_async_copy(yp[p].at[srcb.at[j1]], rows1, sem1).wait()
            pltpu.async_copy(yp[p].at[srcb.at[j2]], rows0, sem0)
            pltpu.sync_copy(rows1, acc.at[dstb.at[j1]], add=True)
            return carry

        lax.fori_loop(0, NCH_L // 2, chunk2, 0)
        pltpu.make_async_copy(yp[p].at[srcb.at[0]], rows0, sem0).wait()
        plsc.subcore_barrier()

        @pl.when(c == 0)
        def _():
            pltpu.sync_copy(acc.at[pl.ds(r0, ROWS_PT)],
                            outs[2 * p].at[pl.ds(r0, ROWS_PT)])

        @pl.when(c == 1)
        def _():
            pltpu.sync_copy(acc.at[pl.ds(r0, ROWS_PT)],
                            outs[2 * p + 1].at[pl.ds(r0, ROWS_PT)])


@functools.cache
def _pool_kernel():
    return functools.partial(
        pl.kernel,
        out_type=[
            jax.ShapeDtypeStruct((GPAD, H), jnp.float32),
            jax.ShapeDtypeStruct((GPAD, H), jnp.float32),
            jax.ShapeDtypeStruct((GPAD, 1), jnp.float32),
            jax.ShapeDtypeStruct((GPAD, 1), jnp.float32),
        ],
        mesh=_sc_mesh(),
        compiler_params=pltpu.CompilerParams(use_tc_tiling_on_sc=False),
        scratch_types=[
            pltpu.VMEM((NCHP, CHP), jnp.int32),
            pltpu.VMEM((NCHP, CHP, 1), jnp.float32),
            pltpu.VMEM((CHP, H), jnp.float32),
            pltpu.VMEM_SHARED((GPAD, H), jnp.float32),
            pltpu.VMEM_SHARED((GPAD, 1), jnp.float32),
        ],
    )(_pool_body)


def _pool_call(x3, bidx, ones_n, zg, zg1):
    return _pool_kernel()(x3, bidx, ones_n, zg, zg1)


def _pool_body(x3, bidx, ones_n, zg, zg1, p0, p1, c0, c1,
               idxb, oneb, valb, acc, cacc):
    c = lax.axis_index("c")
    s = lax.axis_index("s")
    w = c * TILES + s
    base = w * NODES_PT

    @pl.when(s == 0)
    def _():
        pltpu.sync_copy(zg, acc)
        pltpu.sync_copy(zg1, cacc)

    pltpu.sync_copy(bidx.at[w], idxb)
    pltpu.sync_copy(ones_n.at[w], oneb)
    plsc.subcore_barrier()

    def chunk(j, carry):
        pltpu.sync_copy(x3.at[pl.ds(base + j * CHP, CHP)], valb)
        pltpu.sync_copy(valb, acc.at[idxb.at[j]], add=True)
        pltpu.sync_copy(oneb.at[j], cacc.at[idxb.at[j]], add=True)
        return carry

    lax.fori_loop(0, NCHP, chunk, 0)
    plsc.subcore_barrier()

    @pl.when(jnp.logical_and(s == 0, c == 0))
    def _():
        pltpu.sync_copy(acc, p0)
        pltpu.sync_copy(cacc, c0)

    @pl.when(jnp.logical_and(s == 0, c == 1))
    def _():
        pltpu.sync_copy(acc, p1)
        pltpu.sync_copy(cacc, c1)


def _tc_pre_body(x4, d0, d1, w0, *outs):
    ys = outs[:NS]
    dis = outs[NS]
    i = pl.program_id(0)
    deg = d0[...] + d1[...] + 1.0
    row = i * BN + lax.broadcasted_iota(jnp.int32, (BN, 1), 0)
    disv = jnp.where(row < N, lax.rsqrt(deg), 0.0)
    h = jnp.dot(x4[...], w0[...], preferred_element_type=jnp.float32)
    y = disv * h
    for k in range(NS):
        ys[k][...] = y[:, k * HS:(k + 1) * HS]
    dis[...] = disv


def _pre_call(x4, d0, d1, W0):
    sspec = pl.BlockSpec((BN, HS), lambda i: (i, 0))
    return pl.pallas_call(
        _tc_pre_body,
        grid=(GRID,),
        in_specs=[
            pl.BlockSpec((BN, 4), lambda i: (i, 0)),
            pl.BlockSpec((BN, 1), lambda i: (i, 0)),
            pl.BlockSpec((BN, 1), lambda i: (i, 0)),
            pl.BlockSpec((4, H), lambda i: (0, 0)),
        ],
        out_specs=[sspec] * NS + [pl.BlockSpec((BN, 1), lambda i: (i, 0))],
        out_shape=[_F32S] * NS + [jax.ShapeDtypeStruct((NPAD, 1), jnp.float32)],
    )(x4, d0, d1, W0)


def _tc_mid_body(*refs):
    aq = refs[:NS]
    yq = refs[NS:2 * NS]
    dis, b, w = refs[2 * NS:2 * NS + 3]
    outs = refs[2 * NS + 3:]
    oq = outs[:NS]
    xout = outs[NS]
    z = jnp.concatenate([aq[k][...] + yq[k][...] for k in range(NS)], axis=1)
    x = jnp.maximum(dis[...] * z + b[...], 0.0)
    h = jnp.dot(x, w[...], preferred_element_type=jnp.float32)
    y = dis[...] * h
    for k in range(NS):
        oq[k][...] = y[:, k * HS:(k + 1) * HS]
    xout[...] = x


def _mid_call(aq, yq, dis, b, W):
    sspec = pl.BlockSpec((BN, HS), lambda i: (i, 0))
    return pl.pallas_call(
        _tc_mid_body,
        grid=(GRID,),
        in_specs=[sspec] * (2 * NS) + [
            pl.BlockSpec((BN, 1), lambda i: (i, 0)),
            pl.BlockSpec((1, H), lambda i: (0, 0)),
            pl.BlockSpec((H, H), lambda i: (0, 0))],
        out_specs=[sspec] * NS + [pl.BlockSpec((BN, H), lambda i: (i, 0))],
        out_shape=[_F32S] * NS + [jax.ShapeDtypeStruct((NPAD, H), jnp.float32)],
    )(*aq, *yq, dis, b, W)


def _tc_gate_body(p0, p1, c0, c1, wl, bl, out):
    pooled = p0[...] + p1[...]
    cnt = c0[...] + c1[...]
    pooled = pooled[:G] / jnp.maximum(cnt[:G], 1.0)
    logits = jnp.dot(pooled, wl[...], preferred_element_type=jnp.float32) + bl[...]
    m = jnp.max(logits, axis=1, keepdims=True)
    e = jnp.exp(logits - m)
    out[...] = e / jnp.sum(e, axis=1, keepdims=True)


def _gate_call(p0, p1, c0, c1, Wl, bl):
    return pl.pallas_call(
        _tc_gate_body,
        out_shape=jax.ShapeDtypeStruct((G, NX), jnp.float32),
    )(p0, p1, c0, c1, Wl, bl)


def kernel(atomic_numbers, pos, edge_index, batch,
           W0, b0, W1, b1, W2, b2, Wl, bl):
    f32 = jnp.float32
    src = edge_index[0]
    dst = edge_index[1]
    pe = EPAD - E
    srcp = jnp.concatenate([src, jnp.full((pe,), N, jnp.int32)])
    dstp = jnp.concatenate([dst, jnp.full((pe,), N, jnp.int32)])
    src0_l = srcp.reshape(TILES, NCH_L, CH)
    src1_l = (srcp + NPAD).reshape(TILES, NCH_L, CH)
    dst_l = dstp.reshape(TILES, NCH_L, CH)
    dst_d = dstp.reshape(NW, NCH_D, CH)

    batchp = jnp.concatenate(
        [batch.astype(jnp.int32), jnp.full((NPAD - N,), G, jnp.int32)]
    ).reshape(NW, NCHP, CHP)
    ones_n = jnp.concatenate(
        [jnp.ones((N,), f32), jnp.zeros((NPAD - N,), f32)]
    ).reshape(NW, NCHP, CHP, 1)

    x4 = jnp.pad(
        jnp.concatenate([atomic_numbers[:, None], pos], axis=1),
        ((0, NPAD - N), (0, 0)),
    )

    zrows = jnp.zeros((ROWS_PT, HS), f32)
    zcol = jnp.zeros((ROWS_PT, 1), f32)
    ones_ch = jnp.ones((CH, 1), f32)
    zg = jnp.zeros((GPAD, H), f32)
    zg1 = jnp.zeros((GPAD, 1), f32)

    d0, d1 = _deg_call(dst_d, ones_ch, zcol)
    *yq, dis = _pre_call(x4, d0, d1, W0)

    Ws = jnp.stack([W1, W2, W1])
    bs = jnp.stack([b0.reshape(1, H), b1.reshape(1, H), b2.reshape(1, H)])
    trip = lax.optimization_barrier(jnp.int32(3))

    def layer_cond(carry):
        return carry[0] < trip

    def layer_step(carry):
        i, yq, _x = carry
        W = lax.dynamic_index_in_dim(Ws, i, keepdims=False)
        b = lax.dynamic_index_in_dim(bs, i, keepdims=False)
        aq = _agg_call(yq, src0_l, src1_l, dst_l, zrows)
        *oq, x = _mid_call(aq, yq, dis, b, W)
        return (i + 1, tuple(oq), x)

    x_init = jnp.zeros((NPAD, H), f32)
    _, yq, x3 = lax.while_loop(
        layer_cond, layer_step, (jnp.int32(0), tuple(yq), x_init))

    p0, p1, c0, c1 = _pool_call(x3, batchp, ones_n, zg, zg1)
    probs = _gate_call(p0, p1, c0, c1, Wl, bl.reshape(1, NX))
    return probs[:, :, None]
